# Initial kernel scaffold; baseline (speedup 1.0000x reference)
import jax
import jax.numpy as jnp
from jax.experimental import pallas as pl


def kernel(vq_loss, indices, num_embeddings):
    raise NotImplementedError("write your pallas kernel here")


# speedup vs baseline: 1.6310x; 1.6310x over previous
import functools

import jax
import jax.numpy as jnp
from jax import lax
from jax.experimental import pallas as pl
from jax.experimental.pallas import tpu as pltpu
from jax.experimental.pallas import tpu_sc as plsc

_NE = 1024
_NTOK = 16 * 4096
_LANES = 16


def _sc_partial_hist(flat_idx, nc, ns):
  nw = nc * ns
  chunk = _NTOK // nw
  mesh = plsc.VectorSubcoreMesh(core_axis_name="c", subcore_axis_name="s")

  @functools.partial(
      pl.kernel,
      out_type=jax.ShapeDtypeStruct((nw, _NE), jnp.float32),
      mesh=mesh,
      compiler_params=pltpu.CompilerParams(needs_layout_passes=False),
      scratch_types=[
          pltpu.VMEM((chunk,), jnp.int32),
          pltpu.VMEM((_NE,), jnp.float32),
      ],
  )
  def hist(idx_hbm, out_hbm, idx_v, counts_v):
    wid = lax.axis_index("s") * nc + lax.axis_index("c")
    base = wid * chunk
    pltpu.sync_copy(idx_hbm.at[pl.ds(base, chunk)], idx_v)

    zeros = jnp.zeros((_LANES,), jnp.float32)

    def zero_body(i, carry):
      counts_v[pl.ds(i * _LANES, _LANES)] = zeros
      return carry

    lax.fori_loop(0, _NE // _LANES, zero_body, 0, unroll=8)

    ones = jnp.ones((_LANES,), jnp.float32)

    def body(i, carry):
      idx = idx_v[pl.ds(i * _LANES, _LANES)]
      plsc.addupdate_scatter(counts_v, [idx], ones)
      return carry

    lax.fori_loop(0, chunk // _LANES, body, 0, unroll=8)

    pltpu.sync_copy(counts_v, out_hbm.at[wid])

  return hist(flat_idx)


def _finish_body(vq_ref, ne_ref, p_ref, out_ref):
  p = p_ref[...]
  counts = jnp.sum(p, axis=0, keepdims=True)
  usage = counts * (1.0 / _NTOK)
  ent = -jnp.sum(usage * jnp.log(usage + 1e-08))
  util = jnp.mean((usage > 1e-06).astype(jnp.float32))
  max_ent = jnp.sum(jnp.log(jnp.full((1, 128), ne_ref[0], jnp.float32))) * (
      1.0 / 128.0)
  commit = 0.25 * vq_ref[0]
  div = -0.1 * (ent / max_ent)
  out_ref[0] = commit + div
  out_ref[1] = commit
  out_ref[2] = div
  out_ref[3] = util


def kernel(vq_loss, indices, num_embeddings):
  try:
    info = plsc.get_sparse_core_info()
    nc, ns = info.num_cores, info.num_subcores
  except RuntimeError:
    nc, ns = 2, 16
  flat = indices.reshape(-1)
  partials = _sc_partial_hist(flat, nc, ns)

  vq = jnp.asarray(vq_loss, jnp.float32).reshape(1)
  ne = jnp.asarray(num_embeddings, jnp.float32).reshape(1)
  out = pl.pallas_call(
      _finish_body,
      out_shape=jax.ShapeDtypeStruct((4,), jnp.float32),
      in_specs=[
          pl.BlockSpec(memory_space=pltpu.SMEM),
          pl.BlockSpec(memory_space=pltpu.SMEM),
          pl.BlockSpec(memory_space=pltpu.VMEM),
      ],
      out_specs=pl.BlockSpec(memory_space=pltpu.SMEM),
  )(vq, ne, partials)
  return (out[0], out[1], out[2], out[3])
